# baseline (speedup 1.0000x reference)
import functools

import jax
import jax.numpy as jnp
from jax import lax
from jax.experimental import pallas as pl
from jax.experimental.pallas import tpu as pltpu
from jax.experimental.pallas import tpu_sc as plsc

D_MODEL = 1024
B_TOTAL = 8192
NUM_CORES = 2
NUM_SUBCORES = 16
NW = NUM_CORES * NUM_SUBCORES
B_PER_W = B_TOTAL // NW
CHUNK = 16
NCHUNK = B_PER_W // CHUNK
NBUF = 4


def _build():
    mesh = plsc.VectorSubcoreMesh(core_axis_name="c", subcore_axis_name="s")

    @functools.partial(
        pl.kernel,
        mesh=mesh,
        out_type=jax.ShapeDtypeStruct((4, 2048, D_MODEL), jnp.float32),
        scratch_types=[
            pltpu.VMEM((B_PER_W,), jnp.int32),
        ]
        + [pltpu.VMEM((CHUNK, D_MODEL), jnp.float32) for _ in range(NBUF)]
        + [pltpu.SemaphoreType.DMA for _ in range(2 * NBUF)],
    )
    def emb(idx_hbm, table_hbm, out_hbm, idx_v, *scratch):
        bufs = scratch[:NBUF]
        gsems = scratch[NBUF:2 * NBUF]
        wsems = scratch[2 * NBUF:]
        wid = lax.axis_index("s") * NUM_CORES + lax.axis_index("c")
        base = wid * B_PER_W
        w_per_row = 2048 // B_PER_W
        pltpu.sync_copy(
            idx_hbm.at[wid // w_per_row,
                       pl.ds((wid % w_per_row) * B_PER_W, B_PER_W)],
            idx_v,
        )

        gathers = [None] * NBUF
        writes = [None] * NBUF
        for j in range(NBUF - 1):
            gathers[j] = pltpu.async_copy(
                table_hbm.at[idx_v.at[pl.ds(j * CHUNK, CHUNK)]], bufs[j], gsems[j]
            )
        for j in range(NCHUNK):
            b = j % NBUF
            nj = j + NBUF - 1
            if nj < NCHUNK:
                bn = nj % NBUF
                if writes[bn] is not None:
                    writes[bn].wait()
                gathers[bn] = pltpu.async_copy(
                    table_hbm.at[idx_v.at[pl.ds(nj * CHUNK, CHUNK)]],
                    bufs[bn],
                    gsems[bn],
                )
            gathers[b].wait()
            flat = base + j * CHUNK
            writes[b] = pltpu.async_copy(
                bufs[b],
                out_hbm.at[flat // 2048, pl.ds(flat % 2048, CHUNK)],
                wsems[b],
            )
        for j in range(NCHUNK - NBUF, NCHUNK):
            writes[j % NBUF].wait()

    return emb


_emb = _build()


@jax.jit
def kernel(input_ids, weight):
    ids = input_ids
    if ids.dtype != jnp.int32:
        ids = ids.astype(jnp.int32)
    return _emb(ids, weight)


# speedup vs baseline: 1.0370x; 1.0370x over previous
import functools

import jax
import jax.numpy as jnp
from jax import lax
from jax.experimental import pallas as pl
from jax.experimental.pallas import tpu as pltpu
from jax.experimental.pallas import tpu_sc as plsc

D_MODEL = 1024
B_TOTAL = 8192
NUM_CORES = 2
NUM_SUBCORES = 16
NW = NUM_CORES * NUM_SUBCORES
B_PER_W = B_TOTAL // NW
CHUNK = 16
NCHUNK = B_PER_W // CHUNK
NBUF = 6


def _build():
    mesh = plsc.VectorSubcoreMesh(core_axis_name="c", subcore_axis_name="s")

    @functools.partial(
        pl.kernel,
        mesh=mesh,
        out_type=jax.ShapeDtypeStruct((4, 2048, D_MODEL), jnp.float32),
        scratch_types=[
            pltpu.VMEM((B_PER_W,), jnp.int32),
        ]
        + [pltpu.VMEM((CHUNK, D_MODEL), jnp.float32) for _ in range(NBUF)]
        + [pltpu.SemaphoreType.DMA for _ in range(2 * NBUF)],
    )
    def emb(idx_hbm, table_hbm, out_hbm, idx_v, *scratch):
        bufs = scratch[:NBUF]
        gsems = scratch[NBUF:2 * NBUF]
        wsems = scratch[2 * NBUF:]
        wid = lax.axis_index("s") * NUM_CORES + lax.axis_index("c")
        base = wid * B_PER_W
        w_per_row = 2048 // B_PER_W
        pltpu.sync_copy(
            idx_hbm.at[wid // w_per_row,
                       pl.ds((wid % w_per_row) * B_PER_W, B_PER_W)],
            idx_v,
        )

        gathers = [None] * NBUF
        writes = [None] * NBUF
        for j in range(NBUF - 1):
            gathers[j] = pltpu.async_copy(
                table_hbm.at[idx_v.at[pl.ds(j * CHUNK, CHUNK)]], bufs[j], gsems[j]
            )
        for j in range(NCHUNK):
            b = j % NBUF
            nj = j + NBUF - 1
            if nj < NCHUNK:
                bn = nj % NBUF
                if writes[bn] is not None:
                    writes[bn].wait()
                gathers[bn] = pltpu.async_copy(
                    table_hbm.at[idx_v.at[pl.ds(nj * CHUNK, CHUNK)]],
                    bufs[bn],
                    gsems[bn],
                )
            gathers[b].wait()
            flat = base + j * CHUNK
            writes[b] = pltpu.async_copy(
                bufs[b],
                out_hbm.at[flat // 2048, pl.ds(flat % 2048, CHUNK)],
                wsems[b],
            )
        for j in range(NCHUNK - NBUF, NCHUNK):
            writes[j % NBUF].wait()

    return emb


_emb = _build()


@jax.jit
def kernel(input_ids, weight):
    ids = input_ids
    if ids.dtype != jnp.int32:
        ids = ids.astype(jnp.int32)
    return _emb(ids, weight)
